# Initial kernel scaffold; baseline (speedup 1.0000x reference)
import jax
import jax.numpy as jnp
from jax.experimental import pallas as pl


def kernel(f_feat, b_feat, Wq, bq, Wk, bk, Wg, bg, gamma):
    raise NotImplementedError("write your pallas kernel here")


# speedup vs baseline: 54.5653x; 54.5653x over previous
import jax
import jax.numpy as jnp
from jax.experimental import pallas as pl
from jax.experimental.pallas import tpu as pltpu

N, C, H, W = 4, 128, 48, 48
HW = H * W
OC = C // 2
COLT = 1152
T = HW // COLT
K = 5
_PREC = jax.lax.Precision.DEFAULT


def _gn_kernel(bt_ref, ft_ref, f2_ref, wqt_ref, wk_ref, wga_ref,
               bq_ref, bk_ref, bga_ref, gam_ref, out_ref,
               q_s, qn_s, acc_s):
    t = pl.program_id(1)

    @pl.when(t == 0)
    def _init():
        bt = bt_ref[0]
        q = jnp.dot(bt, wqt_ref[...], preferred_element_type=jnp.float32,
                    precision=_PREC) + bq_ref[0:1, :]
        q_s[...] = q
        nrm = jnp.sqrt(jnp.sum(q * q, axis=1, keepdims=True))
        qn_s[...] = q / jnp.maximum(nrm, 1e-12)
        acc_s[...] = jnp.zeros_like(acc_s)

    kq = jnp.dot(wk_ref[...], f2_ref[0], preferred_element_type=jnp.float32,
                 precision=_PREC) + bk_ref[:, 0:1]
    knrm = jnp.sqrt(jnp.sum(kq * kq, axis=0, keepdims=True))
    kn = kq / jnp.maximum(knrm, 1e-12)
    xa = jnp.dot(ft_ref[0], wga_ref[...], preferred_element_type=jnp.float32,
                 precision=_PREC) + bga_ref[0:1, :]

    cur = jnp.clip(jnp.dot(qn_s[...], kn, preferred_element_type=jnp.float32,
                           precision=_PREC), -1.0, 1.0)
    M = jnp.zeros((HW, COLT), jnp.float32)
    for _ in range(K):
        m = jnp.max(cur, axis=0, keepdims=True)
        hit = cur == m
        M = jnp.where(hit, jnp.exp(2.0 * (m - 1.0)), M)
        cur = jnp.where(hit, -3.0, cur)
    acc_s[...] += jnp.dot(M, xa, preferred_element_type=jnp.float32,
                          precision=_PREC)

    @pl.when(t == T - 1)
    def _fin():
        d = jnp.maximum(acc_s[:, OC:OC + 1], 1e-12)
        out_ref[0] = gam_ref[0:1, 0:1] * acc_s[:, :OC] / d + q_s[...]


@jax.jit
def kernel(f_feat, b_feat, Wq, bq, Wk, bk, Wg, bg, gamma):
    f2 = f_feat.reshape(N, C, HW)
    ft = jnp.swapaxes(f2, 1, 2)
    bt = jnp.swapaxes(b_feat.reshape(N, C, HW), 1, 2)
    wga = jnp.concatenate([Wg.T, jnp.zeros((C, C - OC), jnp.float32)], axis=1)
    bga = jnp.concatenate([bg, jnp.ones((1,), jnp.float32),
                           jnp.zeros((C - OC - 1,), jnp.float32)])
    bq2 = jnp.broadcast_to(bq[None, :], (8, OC))
    bk2 = jnp.broadcast_to(bk[:, None], (OC, 128))
    bga2 = jnp.broadcast_to(bga[None, :], (8, C))
    gam = jnp.broadcast_to(gamma[None, :], (8, 128))

    out = pl.pallas_call(
        _gn_kernel,
        grid=(N, T),
        in_specs=[
            pl.BlockSpec((1, HW, C), lambda n, t: (n, 0, 0)),
            pl.BlockSpec((1, COLT, C), lambda n, t: (n, t, 0)),
            pl.BlockSpec((1, C, COLT), lambda n, t: (n, 0, t)),
            pl.BlockSpec((C, OC), lambda n, t: (0, 0)),
            pl.BlockSpec((OC, C), lambda n, t: (0, 0)),
            pl.BlockSpec((C, C), lambda n, t: (0, 0)),
            pl.BlockSpec((8, OC), lambda n, t: (0, 0)),
            pl.BlockSpec((OC, 128), lambda n, t: (0, 0)),
            pl.BlockSpec((8, C), lambda n, t: (0, 0)),
            pl.BlockSpec((8, 128), lambda n, t: (0, 0)),
        ],
        out_specs=pl.BlockSpec((1, HW, OC), lambda n, t: (n, 0, 0)),
        out_shape=jax.ShapeDtypeStruct((N, HW, OC), jnp.float32),
        scratch_shapes=[
            pltpu.VMEM((HW, OC), jnp.float32),
            pltpu.VMEM((HW, OC), jnp.float32),
            pltpu.VMEM((HW, C), jnp.float32),
        ],
    )(bt, ft, f2, Wq.T, Wk, wga, bq2, bk2, bga2, gam)

    return jnp.swapaxes(out, 1, 2).reshape(N, OC, H, W)


# speedup vs baseline: 65.6479x; 1.2031x over previous
import jax
import jax.numpy as jnp
from jax.experimental import pallas as pl
from jax.experimental.pallas import tpu as pltpu

N, C, H, W = 4, 128, 48, 48
HW = H * W
OC = C // 2
COLT = 1152
T = HW // COLT
K = 5
_PREC = jax.lax.Precision.DEFAULT


def _gn_kernel(bt_ref, ft_ref, f2_ref, wqt_ref, wk_ref, wga_ref,
               bq_ref, bk_ref, bga_ref, gam_ref, out_ref,
               q_s, qn_s, acc_s):
    t = pl.program_id(1)

    @pl.when(t == 0)
    def _init():
        bt = bt_ref[0]
        q = jnp.dot(bt, wqt_ref[...], preferred_element_type=jnp.float32,
                    precision=_PREC) + bq_ref[0:1, :]
        q_s[...] = q
        nrm = jnp.sqrt(jnp.sum(q * q, axis=1, keepdims=True))
        qn_s[...] = q / jnp.maximum(nrm, 1e-12)
        acc_s[...] = jnp.zeros_like(acc_s)

    kq = jnp.dot(wk_ref[...], f2_ref[0], preferred_element_type=jnp.float32,
                 precision=_PREC) + bk_ref[:, 0:1]
    knrm = jnp.sqrt(jnp.sum(kq * kq, axis=0, keepdims=True))
    kn = kq / jnp.maximum(knrm, 1e-12)
    xa = jnp.dot(ft_ref[0], wga_ref[...], preferred_element_type=jnp.float32,
                 precision=_PREC) + bga_ref[0:1, :]

    cur = jnp.clip(jnp.dot(qn_s[...], kn, preferred_element_type=jnp.float32,
                           precision=_PREC), -1.0, 1.0)
    for _ in range(K):
        m = jnp.max(cur, axis=0, keepdims=True)
        mv = -jnp.exp(2.0 * (m - 1.0)) - 2.0
        cur = jnp.where(cur == m, mv, cur)
    M = jnp.where(cur < -1.5, -2.0 - cur, 0.0)
    acc_s[...] += jnp.dot(M, xa, preferred_element_type=jnp.float32,
                          precision=_PREC)

    @pl.when(t == T - 1)
    def _fin():
        d = jnp.maximum(acc_s[:, OC:OC + 1], 1e-12)
        out_ref[0] = gam_ref[0:1, 0:1] * acc_s[:, :OC] / d + q_s[...]


@jax.jit
def kernel(f_feat, b_feat, Wq, bq, Wk, bk, Wg, bg, gamma):
    f2 = f_feat.reshape(N, C, HW)
    ft = jnp.swapaxes(f2, 1, 2)
    bt = jnp.swapaxes(b_feat.reshape(N, C, HW), 1, 2)
    wga = jnp.concatenate([Wg.T, jnp.zeros((C, C - OC), jnp.float32)], axis=1)
    bga = jnp.concatenate([bg, jnp.ones((1,), jnp.float32),
                           jnp.zeros((C - OC - 1,), jnp.float32)])
    bq2 = jnp.broadcast_to(bq[None, :], (8, OC))
    bk2 = jnp.broadcast_to(bk[:, None], (OC, 128))
    bga2 = jnp.broadcast_to(bga[None, :], (8, C))
    gam = jnp.broadcast_to(gamma[None, :], (8, 128))

    out = pl.pallas_call(
        _gn_kernel,
        grid=(N, T),
        in_specs=[
            pl.BlockSpec((1, HW, C), lambda n, t: (n, 0, 0)),
            pl.BlockSpec((1, COLT, C), lambda n, t: (n, t, 0)),
            pl.BlockSpec((1, C, COLT), lambda n, t: (n, 0, t)),
            pl.BlockSpec((C, OC), lambda n, t: (0, 0)),
            pl.BlockSpec((OC, C), lambda n, t: (0, 0)),
            pl.BlockSpec((C, C), lambda n, t: (0, 0)),
            pl.BlockSpec((8, OC), lambda n, t: (0, 0)),
            pl.BlockSpec((OC, 128), lambda n, t: (0, 0)),
            pl.BlockSpec((8, C), lambda n, t: (0, 0)),
            pl.BlockSpec((8, 128), lambda n, t: (0, 0)),
        ],
        out_specs=pl.BlockSpec((1, HW, OC), lambda n, t: (n, 0, 0)),
        out_shape=jax.ShapeDtypeStruct((N, HW, OC), jnp.float32),
        scratch_shapes=[
            pltpu.VMEM((HW, OC), jnp.float32),
            pltpu.VMEM((HW, OC), jnp.float32),
            pltpu.VMEM((HW, C), jnp.float32),
        ],
    )(bt, ft, f2, Wq.T, Wk, wga, bq2, bk2, bga2, gam)

    return jnp.swapaxes(out, 1, 2).reshape(N, OC, H, W)
